# baseline (speedup 1.0000x reference)
import jax
import jax.numpy as jnp
from jax import lax
from jax.experimental import pallas as pl
from jax.experimental.pallas import tpu as pltpu
from jax.experimental.pallas import tpu_sc as plsc

_D = 64
_B = 4096
_S = 200
_N = _B * _S
_NW = 32
_PER_W = _N // _NW
_G = 128
_CHUNK = 512
_NG = _CHUNK // _G
_NCH = _PER_W // _CHUNK
_SCALE = 8.0


def _body(idx_hbm, tab_hbm, out_hbm, idx_v, rows_v, sem_g):
    c = lax.axis_index("c")
    s = lax.axis_index("s")
    wid = s * 2 + c

    def chunk(k, carry):
        gbase = wid * (_PER_W // _G) + k * _NG
        rbase = wid * _PER_W + k * _CHUNK
        pltpu.sync_copy(idx_hbm.at[pl.ds(gbase, _NG)], idx_v)
        cps = [
            pltpu.async_copy(
                tab_hbm.at[idx_v.at[j]],
                rows_v.at[pl.ds(j * _G, _G)],
                sem_g,
            )
            for j in range(_NG)
        ]
        for cp in cps:
            cp.wait()

        def row(r, rcarry):
            for cc in range(_D // 16):
                sl = pl.ds(cc * 16, 16)
                rows_v[r, sl] = rows_v[r, sl] * _SCALE
            return rcarry

        lax.fori_loop(0, _CHUNK, row, 0, unroll=8)
        pltpu.sync_copy(rows_v, out_hbm.at[pl.ds(rbase, _CHUNK)])
        return carry

    lax.fori_loop(0, _NCH, chunk, 0)


def kernel(vocab_to_embed, embedding_table):
    idx = vocab_to_embed.reshape(_N // _G, _G).astype(jnp.int32)
    mesh = plsc.VectorSubcoreMesh(core_axis_name="c", subcore_axis_name="s")
    f = pl.kernel(
        _body,
        mesh=mesh,
        compiler_params=pltpu.CompilerParams(use_tc_tiling_on_sc=False),
        out_type=jax.ShapeDtypeStruct((_N, _D), jnp.float32),
        scratch_types=[
            pltpu.VMEM((_NG, _G), jnp.int32),
            pltpu.VMEM((_CHUNK, _D), jnp.float32),
            pltpu.SemaphoreType.DMA,
        ],
    )
    out = f(idx, embedding_table)
    return out.reshape(_B, _S, _D)


# speedup vs baseline: 1.0852x; 1.0852x over previous
import jax
import jax.numpy as jnp
from jax import lax
from jax.experimental import pallas as pl
from jax.experimental.pallas import tpu as pltpu
from jax.experimental.pallas import tpu_sc as plsc

_D = 64
_B = 4096
_S = 200
_N = _B * _S
_NW = 32
_PER_W = _N // _NW
_G = 128
_CHUNK = 256
_NG = _CHUNK // _G
_NCH = _PER_W // _CHUNK
_NBUF = 4
_IROWS = _PER_W // _G
_SCALE = 8.0


def _body(idx_hbm, tab_hbm, out_hbm, idx_all, rows, sems_g, sems_o):
    c = lax.axis_index("c")
    s = lax.axis_index("s")
    wid = s * 2 + c
    obase = wid * _PER_W

    pltpu.sync_copy(idx_hbm.at[pl.ds(wid * _IROWS, _IROWS)], idx_all)

    def issue_gather(k, b):
        for j in range(_NG):
            pltpu.async_copy(
                tab_hbm.at[idx_all.at[k * _NG + j]],
                rows.at[b, pl.ds(j * _G, _G)],
                sems_g[b],
            )

    def wait_gather(b):
        pltpu.make_async_copy(
            out_hbm.at[pl.ds(0, _CHUNK)], rows.at[b], sems_g[b]
        ).wait()

    def issue_out(k, b):
        pltpu.async_copy(
            rows.at[b], out_hbm.at[pl.ds(obase + k * _CHUNK, _CHUNK)], sems_o[b]
        )

    def wait_out(k, b):
        pltpu.make_async_copy(
            rows.at[b], out_hbm.at[pl.ds(obase + k * _CHUNK, _CHUNK)], sems_o[b]
        ).wait()

    issue_gather(0, 0)
    issue_gather(1, 1)

    def outer(k4, carry):
        for b in range(_NBUF):
            k = k4 * _NBUF + b

            @pl.when(k >= 2)
            def _():
                wait_out(k - 2, (b + 2) % _NBUF)

            @pl.when(k + 2 < _NCH)
            def _():
                issue_gather(k + 2, (b + 2) % _NBUF)

            wait_gather(b)

            def row(r, rcarry):
                for cc in range(_D // 16):
                    sl = pl.ds(cc * 16, 16)
                    rows[b, r, sl] = rows[b, r, sl] * _SCALE
                return rcarry

            lax.fori_loop(0, _CHUNK, row, 0, unroll=8)
            issue_out(k, b)
        return carry

    lax.fori_loop(0, _NCH // _NBUF, outer, 0)

    wait_out(_NCH - 2, (_NCH - 2) % _NBUF)
    wait_out(_NCH - 1, (_NCH - 1) % _NBUF)


def kernel(vocab_to_embed, embedding_table):
    idx = vocab_to_embed.reshape(_N // _G, _G).astype(jnp.int32)
    mesh = plsc.VectorSubcoreMesh(core_axis_name="c", subcore_axis_name="s")
    f = pl.kernel(
        _body,
        mesh=mesh,
        compiler_params=pltpu.CompilerParams(use_tc_tiling_on_sc=False),
        out_type=jax.ShapeDtypeStruct((_N, _D), jnp.float32),
        scratch_types=[
            pltpu.VMEM((_IROWS, _G), jnp.int32),
            pltpu.VMEM((_NBUF, _CHUNK, _D), jnp.float32),
            [pltpu.SemaphoreType.DMA] * _NBUF,
            [pltpu.SemaphoreType.DMA] * _NBUF,
        ],
    )
    out = f(idx, embedding_table)
    return out.reshape(_B, _S, _D)
